# baseline (speedup 1.0000x reference)
import functools

import jax
import jax.numpy as jnp
from jax import lax
from jax.experimental import pallas as pl
from jax.experimental.pallas import tpu as pltpu
from jax.experimental.pallas import tpu_sc as plsc

B, L, T = 2, 1, 4096
V, D = 4096, 4096
NT = 10
R = NT + T

NC, NS, LANES = 2, 16, 16
NW = NC * NS
ROWS_PER_W = (B * T) // NW
CH = 8
NXFER = ROWS_PER_W // CH

_mesh = plsc.VectorSubcoreMesh(core_axis_name="c", subcore_axis_name="s")


def _dest_rows():
    w = jnp.arange(NW)
    b = w // (NW // B)
    ob = b * R + NT + (w - b * (NW // B)) * ROWS_PER_W
    gather_rows = (ob[:, None] + jnp.arange(ROWS_PER_W)[None, :])
    gather_rows = gather_rows.reshape(NW, NXFER, CH)
    prow = jnp.where(w < B * NT, (w // NT) * R + (w % NT), ob + 1)
    prompt_rows = jnp.concatenate(
        [prow[:, None], ob[:, None] + jnp.arange(1, CH)[None, :]], axis=1)
    return jnp.concatenate(
        [gather_rows, prompt_rows[:, None, :]], axis=1).astype(jnp.int32)


@functools.partial(
    pl.kernel,
    out_type=jax.ShapeDtypeStruct((B * R, D), jnp.float32),
    mesh=_mesh,
    scratch_types=[
        pltpu.VMEM((ROWS_PER_W,), jnp.int32),
        pltpu.VMEM((NXFER + 1, CH), jnp.int32),
        pltpu.VMEM((2, CH, D), jnp.float32),
        pltpu.VMEM((LANES,), jnp.float32),
        pltpu.VMEM((D,), jnp.float32),
        pltpu.VMEM((D,), jnp.float32),
        pltpu.SemaphoreType.DMA,
        pltpu.SemaphoreType.DMA,
        pltpu.SemaphoreType.DMA,
        pltpu.SemaphoreType.DMA,
    ],
)
def _mpt_sc(idx_hbm, table_hbm, sp_hbm, u16_hbm, v_hbm, orows_hbm, out_hbm,
            idx_v, orows_v, gbuf, u_v, v_v, row_v, gsem0, gsem1, osem0, osem1):
    cid = lax.axis_index("c")
    sid = lax.axis_index("s")
    wid = sid * NC + cid
    gsem = (gsem0, gsem1)
    osem = (osem0, osem1)

    pltpu.sync_copy(idx_hbm.at[pl.ds(wid * ROWS_PER_W, ROWS_PER_W)], idx_v)
    pltpu.sync_copy(orows_hbm.at[wid], orows_v)

    @pl.when(wid < B * NT)
    def _prompt():
        n = wid - (wid // NT) * NT
        pltpu.sync_copy(u16_hbm.at[pl.ds(n * LANES, LANES)], u_v)
        pltpu.sync_copy(v_hbm, v_v)
        pltpu.sync_copy(sp_hbm.at[pl.ds(n * D, D)], row_v)
        un = u_v[...]

        def pbody(j, carry):
            s = pl.ds(j * LANES, LANES)
            gbuf[0, 0, s] = un * v_v[s] * row_v[s]
            return carry

        lax.fori_loop(0, D // LANES, pbody, 0)
        pltpu.async_copy(gbuf.at[0], out_hbm.at[orows_v.at[NXFER]],
                         osem[0]).wait()

    def _gather(t, buf):
        src = table_hbm.at[idx_v.at[pl.ds(t * CH, CH)]]
        pltpu.async_copy(src, gbuf.at[buf], gsem[buf])

    _gather(0, 0)

    def body(i, carry):
        for bb in range(2):
            t = 2 * i + bb
            pltpu.make_async_copy(table_hbm.at[pl.ds(0, CH)], gbuf.at[bb],
                                  gsem[bb]).wait()
            pltpu.async_copy(gbuf.at[bb], out_hbm.at[orows_v.at[t]], osem[bb])

            @pl.when(t >= 1)
            def _():
                pltpu.make_async_copy(gbuf.at[1 - bb],
                                      out_hbm.at[pl.ds(0, CH)],
                                      osem[1 - bb]).wait()

            @pl.when(t + 1 < NXFER)
            def _():
                _gather(t + 1, 1 - bb)
        return carry

    lax.fori_loop(0, NXFER // 2, body, 0)
    pltpu.make_async_copy(gbuf.at[1], out_hbm.at[pl.ds(0, CH)],
                          osem[1]).wait()


def kernel(tokens, wte_weight, shared_prompt, u, v):
    idx = tokens.reshape(B * T).astype(jnp.int32)
    u16 = jnp.tile(u.reshape(NT, 1), (1, LANES)).reshape(NT * LANES)
    out = _mpt_sc(idx, wte_weight, shared_prompt.reshape(NT * D), u16,
                  v.reshape(D), _dest_rows())
    return out.reshape(B, R, D)[:, None]


# speedup vs baseline: 2.6176x; 2.6176x over previous
import functools

import jax
import jax.numpy as jnp
from jax import lax
from jax.experimental import pallas as pl
from jax.experimental.pallas import tpu as pltpu
from jax.experimental.pallas import tpu_sc as plsc

B, L, T = 2, 1, 4096
V, D = 4096, 4096
NT = 10
R = NT + T

NC, NS, LANES = 2, 16, 16
NW = NC * NS
ROWS_PER_W = (B * T) // NW
CH = 8
NXFER = ROWS_PER_W // CH

_mesh = plsc.VectorSubcoreMesh(core_axis_name="c", subcore_axis_name="s")


def _dest_rows():
    w = jnp.arange(NW)
    b = w // (NW // B)
    ob = NT + (w - b * (NW // B)) * ROWS_PER_W
    gather_rows = (ob[:, None] + jnp.arange(ROWS_PER_W)[None, :])
    gather_rows = gather_rows.reshape(NW, NXFER, CH)
    prow = jnp.where(w < B * NT, w % NT, ob + 1)
    prompt_rows = jnp.tile(prow[:, None], (1, CH))
    return jnp.concatenate(
        [gather_rows, prompt_rows[:, None, :]], axis=1).astype(jnp.int32)


@functools.partial(
    pl.kernel,
    out_type=jax.ShapeDtypeStruct((B, R, D), jnp.float32),
    mesh=_mesh,
    scratch_types=[
        pltpu.VMEM((ROWS_PER_W,), jnp.int32),
        pltpu.VMEM((NXFER + 1, CH), jnp.int32),
        pltpu.VMEM((2, CH, D), jnp.float32),
        pltpu.VMEM((LANES,), jnp.float32),
        pltpu.VMEM((D,), jnp.float32),
        pltpu.VMEM((D,), jnp.float32),
        pltpu.SemaphoreType.DMA,
        pltpu.SemaphoreType.DMA,
        pltpu.SemaphoreType.DMA,
        pltpu.SemaphoreType.DMA,
    ],
)
def _mpt_sc(idx_hbm, table_hbm, sp_hbm, u16_hbm, v_hbm, orows_hbm, out_hbm,
            idx_v, orows_v, gbuf, u_v, v_v, row_v, gsem0, gsem1, osem0, osem1):
    cid = lax.axis_index("c")
    sid = lax.axis_index("s")
    wid = sid * NC + cid
    gsem = (gsem0, gsem1)
    osem = (osem0, osem1)

    b = wid // (NW // B)
    pltpu.sync_copy(idx_hbm.at[pl.ds(wid * ROWS_PER_W, ROWS_PER_W)], idx_v)
    pltpu.sync_copy(orows_hbm.at[wid], orows_v)
    out_b = out_hbm.at[b]

    @pl.when(wid < B * NT)
    def _prompt():
        bp = wid // NT
        n = wid - bp * NT
        pltpu.sync_copy(u16_hbm.at[pl.ds(n * LANES, LANES)], u_v)
        pltpu.sync_copy(v_hbm, v_v)
        pltpu.sync_copy(sp_hbm.at[pl.ds(n * D, D)], row_v)
        un = u_v[...]

        def pbody(j, carry):
            s = pl.ds(j * LANES, LANES)
            val = un * v_v[s] * row_v[s]
            for k in range(CH):
                gbuf[0, k, s] = val
            return carry

        lax.fori_loop(0, D // LANES, pbody, 0)
        pltpu.async_copy(gbuf.at[0], out_hbm.at[bp].at[orows_v.at[NXFER]],
                         osem[0]).wait()

    def _gather(t, buf):
        src = table_hbm.at[idx_v.at[pl.ds(t * CH, CH)]]
        pltpu.async_copy(src, gbuf.at[buf], gsem[buf])

    _gather(0, 0)

    def body(i, carry):
        for bb in range(2):
            t = 2 * i + bb
            pltpu.make_async_copy(table_hbm.at[pl.ds(0, CH)], gbuf.at[bb],
                                  gsem[bb]).wait()
            pltpu.async_copy(gbuf.at[bb], out_b.at[orows_v.at[t]], osem[bb])

            @pl.when(t >= 1)
            def _():
                pltpu.make_async_copy(gbuf.at[1 - bb],
                                      out_b.at[pl.ds(0, CH)],
                                      osem[1 - bb]).wait()

            @pl.when(t + 1 < NXFER)
            def _():
                _gather(t + 1, 1 - bb)
        return carry

    lax.fori_loop(0, NXFER // 2, body, 0)
    pltpu.make_async_copy(gbuf.at[1], out_b.at[pl.ds(0, CH)],
                          osem[1]).wait()


def kernel(tokens, wte_weight, shared_prompt, u, v):
    idx = tokens.reshape(B * T).astype(jnp.int32)
    u16 = jnp.tile(u.reshape(NT, 1), (1, LANES)).reshape(NT * LANES)
    out = _mpt_sc(idx, wte_weight, shared_prompt.reshape(NT * D), u16,
                  v.reshape(D), _dest_rows())
    return out[:, None]


# speedup vs baseline: 3.3840x; 1.2928x over previous
import functools

import jax
import jax.numpy as jnp
from jax import lax
from jax.experimental import pallas as pl
from jax.experimental.pallas import tpu as pltpu
from jax.experimental.pallas import tpu_sc as plsc

B, L, T = 2, 1, 4096
V, D = 4096, 4096
NT = 10
R = NT + T

NC, NS, LANES = 2, 16, 16
NW = NC * NS
ROWS_PER_W = (B * T) // NW
CH = 8
NXFER = ROWS_PER_W // CH

_mesh = plsc.VectorSubcoreMesh(core_axis_name="c", subcore_axis_name="s")


def _dest_rows():
    w = jnp.arange(NW)
    b = w // (NW // B)
    ob = NT + (w - b * (NW // B)) * ROWS_PER_W
    gather_rows = (ob[:, None] + jnp.arange(ROWS_PER_W)[None, :])
    gather_rows = gather_rows.reshape(NW, NXFER, CH)
    prow = jnp.where(w < B * NT, w % NT, ob + 1)
    prompt_rows = jnp.tile(prow[:, None], (1, CH))
    return jnp.concatenate(
        [gather_rows, prompt_rows[:, None, :]], axis=1).astype(jnp.int32)


@functools.partial(
    pl.kernel,
    out_type=jax.ShapeDtypeStruct((B, R, D), jnp.float32),
    mesh=_mesh,
    compiler_params=pltpu.CompilerParams(use_tc_tiling_on_sc=False),
    scratch_types=[
        pltpu.VMEM((ROWS_PER_W,), jnp.int32),
        pltpu.VMEM((NXFER + 1, CH), jnp.int32),
        pltpu.VMEM((2, CH, D), jnp.float32),
        pltpu.VMEM((LANES,), jnp.float32),
        pltpu.VMEM((D,), jnp.float32),
        pltpu.VMEM((D,), jnp.float32),
        pltpu.SemaphoreType.DMA,
        pltpu.SemaphoreType.DMA,
        pltpu.SemaphoreType.DMA,
        pltpu.SemaphoreType.DMA,
    ],
)
def _mpt_sc(idx_hbm, table_hbm, sp_hbm, u16_hbm, v_hbm, orows_hbm, out_hbm,
            idx_v, orows_v, gbuf, u_v, v_v, row_v, gsem0, gsem1, osem0, osem1):
    cid = lax.axis_index("c")
    sid = lax.axis_index("s")
    wid = sid * NC + cid
    gsem = (gsem0, gsem1)
    osem = (osem0, osem1)

    b = wid // (NW // B)
    pltpu.sync_copy(idx_hbm.at[pl.ds(wid * ROWS_PER_W, ROWS_PER_W)], idx_v)
    pltpu.sync_copy(orows_hbm.at[wid], orows_v)
    out_b = out_hbm.at[b]

    @pl.when(wid < B * NT)
    def _prompt():
        bp = wid // NT
        n = wid - bp * NT
        pltpu.sync_copy(u16_hbm.at[pl.ds(n * LANES, LANES)], u_v)
        pltpu.sync_copy(v_hbm, v_v)
        pltpu.sync_copy(sp_hbm.at[pl.ds(n * D, D)], row_v)
        un = u_v[...]

        def pbody(j, carry):
            s = pl.ds(j * LANES, LANES)
            val = un * v_v[s] * row_v[s]
            for k in range(CH):
                gbuf[0, k, s] = val
            return carry

        lax.fori_loop(0, D // LANES, pbody, 0)
        pltpu.async_copy(gbuf.at[0], out_hbm.at[bp].at[orows_v.at[NXFER]],
                         osem[0]).wait()

    def _gather(t, buf):
        src = table_hbm.at[idx_v.at[pl.ds(t * CH, CH)]]
        pltpu.async_copy(src, gbuf.at[buf], gsem[buf])

    _gather(0, 0)

    def body(i, carry):
        for bb in range(2):
            t = 2 * i + bb
            pltpu.make_async_copy(table_hbm.at[pl.ds(0, CH)], gbuf.at[bb],
                                  gsem[bb]).wait()
            pltpu.async_copy(gbuf.at[bb], out_b.at[orows_v.at[t]], osem[bb])

            @pl.when(t >= 1)
            def _():
                pltpu.make_async_copy(gbuf.at[1 - bb],
                                      out_b.at[pl.ds(0, CH)],
                                      osem[1 - bb]).wait()

            @pl.when(t + 1 < NXFER)
            def _():
                _gather(t + 1, 1 - bb)
        return carry

    lax.fori_loop(0, NXFER // 2, body, 0)
    pltpu.make_async_copy(gbuf.at[1], out_b.at[pl.ds(0, CH)],
                          osem[1]).wait()


def kernel(tokens, wte_weight, shared_prompt, u, v):
    idx = tokens.reshape(B * T).astype(jnp.int32)
    u16 = jnp.tile(u.reshape(NT, 1), (1, LANES)).reshape(NT * LANES)
    out = _mpt_sc(idx, wte_weight, shared_prompt.reshape(NT * D), u16,
                  v.reshape(D), _dest_rows())
    return out[:, None]


# speedup vs baseline: 3.3851x; 1.0003x over previous
import functools

import jax
import jax.numpy as jnp
from jax import lax
from jax.experimental import pallas as pl
from jax.experimental.pallas import tpu as pltpu
from jax.experimental.pallas import tpu_sc as plsc

B, L, T = 2, 1, 4096
V, D = 4096, 4096
NT = 10
R = NT + T

NC, NS, LANES = 2, 16, 16
NW = NC * NS
ROWS_PER_W = (B * T) // NW
CH = 8
NXFER = ROWS_PER_W // CH

_mesh = plsc.VectorSubcoreMesh(core_axis_name="c", subcore_axis_name="s")


def _dest_rows():
    w = jnp.arange(NW)
    b = w // (NW // B)
    ob = NT + (w - b * (NW // B)) * ROWS_PER_W
    gather_rows = (ob[:, None] + jnp.arange(ROWS_PER_W)[None, :])
    gather_rows = gather_rows.reshape(NW, NXFER, CH)
    prow = jnp.where(w < B * NT, w % NT, ob + 1)
    prompt_rows = jnp.tile(prow[:, None], (1, CH))
    return jnp.concatenate(
        [gather_rows, prompt_rows[:, None, :]], axis=1).astype(jnp.int32)


@functools.partial(
    pl.kernel,
    out_type=jax.ShapeDtypeStruct((B, R, D), jnp.float32),
    mesh=_mesh,
    compiler_params=pltpu.CompilerParams(use_tc_tiling_on_sc=False),
    scratch_types=[
        pltpu.VMEM((ROWS_PER_W,), jnp.int32),
        pltpu.VMEM((NXFER + 1, CH), jnp.int32),
        pltpu.VMEM((2, CH, D), jnp.float32),
        pltpu.VMEM((LANES,), jnp.float32),
        pltpu.VMEM((D,), jnp.float32),
        pltpu.VMEM((D,), jnp.float32),
        pltpu.SemaphoreType.DMA,
        pltpu.SemaphoreType.DMA,
        pltpu.SemaphoreType.DMA,
        pltpu.SemaphoreType.DMA,
    ],
)
def _mpt_sc(idx_hbm, table_hbm, sp_hbm, u16_hbm, v_hbm, orows_hbm, out_hbm,
            idx_v, orows_v, gbuf, u_v, v_v, row_v, gsem0, gsem1, osem0, osem1):
    cid = lax.axis_index("c")
    sid = lax.axis_index("s")
    wid = sid * NC + cid
    gsem = (gsem0, gsem1)
    osem = (osem0, osem1)

    b = wid // (NW // B)
    pltpu.sync_copy(idx_hbm.at[pl.ds(wid * ROWS_PER_W, ROWS_PER_W)], idx_v)
    pltpu.sync_copy(orows_hbm.at[wid], orows_v)
    out_b = out_hbm.at[b]
    ob = NT + (wid - b * (NW // B)) * ROWS_PER_W

    @pl.when(wid < B * NT)
    def _prompt():
        bp = wid // NT
        n = wid - bp * NT
        pltpu.sync_copy(u16_hbm.at[pl.ds(n * LANES, LANES)], u_v)
        pltpu.sync_copy(v_hbm, v_v)
        pltpu.sync_copy(sp_hbm.at[pl.ds(n * D, D)], row_v)
        un = u_v[...]

        def pbody(j, carry):
            s = pl.ds(j * LANES, LANES)
            val = un * v_v[s] * row_v[s]
            for k in range(CH):
                gbuf[0, k, s] = val
            return carry

        lax.fori_loop(0, D // LANES, pbody, 0)
        pltpu.async_copy(gbuf.at[0], out_hbm.at[bp].at[orows_v.at[NXFER]],
                         osem[0]).wait()

    def _gather(t, buf):
        src = table_hbm.at[idx_v.at[pl.ds(t * CH, CH)]]
        pltpu.async_copy(src, gbuf.at[buf], gsem[buf])

    _gather(0, 0)

    def body(i, carry):
        for bb in range(2):
            t = 2 * i + bb
            pltpu.make_async_copy(table_hbm.at[pl.ds(0, CH)], gbuf.at[bb],
                                  gsem[bb]).wait()
            pltpu.async_copy(gbuf.at[bb], out_b.at[pl.ds(ob + t * CH, CH)], osem[bb])

            @pl.when(t >= 1)
            def _():
                pltpu.make_async_copy(gbuf.at[1 - bb],
                                      out_b.at[pl.ds(0, CH)],
                                      osem[1 - bb]).wait()

            @pl.when(t + 1 < NXFER)
            def _():
                _gather(t + 1, 1 - bb)
        return carry

    lax.fori_loop(0, NXFER // 2, body, 0)
    pltpu.make_async_copy(gbuf.at[1], out_b.at[pl.ds(0, CH)],
                          osem[1]).wait()


def kernel(tokens, wte_weight, shared_prompt, u, v):
    idx = tokens.reshape(B * T).astype(jnp.int32)
    u16 = jnp.tile(u.reshape(NT, 1), (1, LANES)).reshape(NT * LANES)
    out = _mpt_sc(idx, wte_weight, shared_prompt.reshape(NT * D), u16,
                  v.reshape(D), _dest_rows())
    return out[:, None]


# speedup vs baseline: 3.4049x; 1.0058x over previous
import functools

import jax
import jax.numpy as jnp
from jax import lax
from jax.experimental import pallas as pl
from jax.experimental.pallas import tpu as pltpu
from jax.experimental.pallas import tpu_sc as plsc

B, L, T = 2, 1, 4096
V, D = 4096, 4096
NT = 10
R = NT + T

NC, NS, LANES = 2, 16, 16
NW = NC * NS
ROWS_PER_W = (B * T) // NW
CH = 8
NXFER = ROWS_PER_W // CH

_mesh = plsc.VectorSubcoreMesh(core_axis_name="c", subcore_axis_name="s")


def _dest_rows():
    w = jnp.arange(NW)
    b = w // (NW // B)
    ob = NT + (w - b * (NW // B)) * ROWS_PER_W
    gather_rows = (ob[:, None] + jnp.arange(ROWS_PER_W)[None, :])
    gather_rows = gather_rows.reshape(NW, NXFER, CH)
    prow = jnp.where(w < B * NT, w % NT, ob + 1)
    prompt_rows = jnp.tile(prow[:, None], (1, CH))
    return jnp.concatenate(
        [gather_rows, prompt_rows[:, None, :]], axis=1).astype(jnp.int32)


@functools.partial(
    pl.kernel,
    out_type=jax.ShapeDtypeStruct((B, R, D), jnp.float32),
    mesh=_mesh,
    compiler_params=pltpu.CompilerParams(use_tc_tiling_on_sc=False),
    scratch_types=[
        pltpu.VMEM((ROWS_PER_W,), jnp.int32),
        pltpu.VMEM((NXFER + 1, CH), jnp.int32),
        pltpu.VMEM((3, CH, D), jnp.float32),
        pltpu.VMEM((LANES,), jnp.float32),
        pltpu.VMEM((D,), jnp.float32),
        pltpu.VMEM((D,), jnp.float32),
        pltpu.SemaphoreType.DMA,
        pltpu.SemaphoreType.DMA,
        pltpu.SemaphoreType.DMA,
        pltpu.SemaphoreType.DMA,
        pltpu.SemaphoreType.DMA,
        pltpu.SemaphoreType.DMA,
    ],
)
def _mpt_sc(idx_hbm, table_hbm, sp_hbm, u16_hbm, v_hbm, orows_hbm, out_hbm,
            idx_v, orows_v, gbuf, u_v, v_v, row_v,
            gsem0, gsem1, gsem2, osem0, osem1, osem2):
    cid = lax.axis_index("c")
    sid = lax.axis_index("s")
    wid = sid * NC + cid
    gsem = (gsem0, gsem1, gsem2)
    osem = (osem0, osem1, osem2)

    b = wid // (NW // B)
    pltpu.sync_copy(idx_hbm.at[pl.ds(wid * ROWS_PER_W, ROWS_PER_W)], idx_v)
    pltpu.sync_copy(orows_hbm.at[wid], orows_v)
    out_b = out_hbm.at[b]
    ob = NT + (wid - b * (NW // B)) * ROWS_PER_W

    @pl.when(wid < B * NT)
    def _prompt():
        bp = wid // NT
        n = wid - bp * NT
        pltpu.sync_copy(u16_hbm.at[pl.ds(n * LANES, LANES)], u_v)
        pltpu.sync_copy(v_hbm, v_v)
        pltpu.sync_copy(sp_hbm.at[pl.ds(n * D, D)], row_v)
        un = u_v[...]

        def pbody(j, carry):
            s = pl.ds(j * LANES, LANES)
            val = un * v_v[s] * row_v[s]
            for k in range(CH):
                gbuf[0, k, s] = val
            return carry

        lax.fori_loop(0, D // LANES, pbody, 0)
        pltpu.async_copy(gbuf.at[0], out_hbm.at[bp].at[orows_v.at[NXFER]],
                         osem[0]).wait()

    def _gather(t, buf):
        src = table_hbm.at[idx_v.at[pl.ds(t * CH, CH)]]
        pltpu.async_copy(src, gbuf.at[buf], gsem[buf])

    _gather(0, 0)
    _gather(1, 1)
    for t in range(NXFER):
        bb = t % 3
        pltpu.make_async_copy(table_hbm.at[pl.ds(0, CH)], gbuf.at[bb],
                              gsem[bb]).wait()
        pltpu.async_copy(gbuf.at[bb], out_b.at[pl.ds(ob + t * CH, CH)],
                         osem[bb])
        if t >= 1:
            pltpu.make_async_copy(gbuf.at[(t - 1) % 3],
                                  out_b.at[pl.ds(0, CH)],
                                  osem[(t - 1) % 3]).wait()
        if t + 2 < NXFER:
            _gather(t + 2, (t + 2) % 3)
    pltpu.make_async_copy(gbuf.at[(NXFER - 1) % 3], out_b.at[pl.ds(0, CH)],
                          osem[(NXFER - 1) % 3]).wait()


def kernel(tokens, wte_weight, shared_prompt, u, v):
    idx = tokens.reshape(B * T).astype(jnp.int32)
    u16 = jnp.tile(u.reshape(NT, 1), (1, LANES)).reshape(NT * LANES)
    out = _mpt_sc(idx, wte_weight, shared_prompt.reshape(NT * D), u16,
                  v.reshape(D), _dest_rows())
    return out[:, None]


# speedup vs baseline: 3.4450x; 1.0118x over previous
import functools

import jax
import jax.numpy as jnp
from jax import lax
from jax.experimental import pallas as pl
from jax.experimental.pallas import tpu as pltpu
from jax.experimental.pallas import tpu_sc as plsc

B, L, T = 2, 1, 4096
V, D = 4096, 4096
NT = 10
R = NT + T

NC, NS, LANES = 2, 16, 16
NW = NC * NS
ROWS_PER_W = (B * T) // NW
CH = 8
NBUF = 3
NXFER = ROWS_PER_W // CH

_mesh = plsc.VectorSubcoreMesh(core_axis_name="c", subcore_axis_name="s")


@functools.partial(
    pl.kernel,
    out_type=jax.ShapeDtypeStruct((B, R, D), jnp.float32),
    mesh=_mesh,
    compiler_params=pltpu.CompilerParams(use_tc_tiling_on_sc=False),
    scratch_types=[
        pltpu.VMEM((ROWS_PER_W,), jnp.int32),
        pltpu.VMEM((NBUF, CH, D), jnp.float32),
        pltpu.VMEM((LANES,), jnp.float32),
        pltpu.VMEM((D,), jnp.float32),
        pltpu.VMEM((D,), jnp.float32),
        pltpu.SemaphoreType.DMA,
        pltpu.SemaphoreType.DMA,
        pltpu.SemaphoreType.DMA,
        pltpu.SemaphoreType.DMA,
        pltpu.SemaphoreType.DMA,
        pltpu.SemaphoreType.DMA,
    ],
)
def _mpt_sc(idx_hbm, table_hbm, sp_hbm, u16_hbm, v_hbm, out_hbm,
            idx_v, gbuf, u_v, v_v, row_v,
            gsem0, gsem1, gsem2, osem0, osem1, osem2):
    cid = lax.axis_index("c")
    sid = lax.axis_index("s")
    wid = sid * NC + cid
    gsem = (gsem0, gsem1, gsem2)
    osem = (osem0, osem1, osem2)

    b = wid // (NW // B)
    ob = NT + (wid - b * (NW // B)) * ROWS_PER_W
    out_b = out_hbm.at[b]

    pltpu.sync_copy(idx_hbm.at[pl.ds(wid * ROWS_PER_W, ROWS_PER_W)], idx_v)

    def _gather(t, buf):
        src = table_hbm.at[idx_v.at[pl.ds(t * CH, CH)]]
        pltpu.async_copy(src, gbuf.at[buf], gsem[buf])

    _gather(0, 0)
    _gather(1, 1)

    @pl.when(wid < B * NT)
    def _prompt():
        bp = wid // NT
        n = wid - bp * NT
        pltpu.sync_copy(u16_hbm.at[pl.ds(n * LANES, LANES)], u_v)
        pltpu.sync_copy(v_hbm, v_v)
        pltpu.sync_copy(sp_hbm.at[pl.ds(n * D, D)], row_v)
        un = u_v[...]

        def pbody(j, carry):
            s = pl.ds(j * LANES, LANES)
            gbuf[2, 0, s] = un * v_v[s] * row_v[s]
            return carry

        lax.fori_loop(0, D // LANES, pbody, 0)
        pltpu.async_copy(gbuf.at[2].at[pl.ds(0, 1)],
                         out_hbm.at[bp].at[pl.ds(n, 1)], osem[2]).wait()

    for t in range(NXFER):
        bb = t % NBUF
        pltpu.make_async_copy(table_hbm.at[pl.ds(0, CH)], gbuf.at[bb],
                              gsem[bb]).wait()
        pltpu.async_copy(gbuf.at[bb], out_b.at[pl.ds(ob + t * CH, CH)],
                         osem[bb])
        if t >= 1:
            pltpu.make_async_copy(gbuf.at[(t - 1) % NBUF],
                                  out_b.at[pl.ds(0, CH)],
                                  osem[(t - 1) % NBUF]).wait()
        if t + 2 < NXFER:
            _gather(t + 2, (t + 2) % NBUF)
    pltpu.make_async_copy(gbuf.at[(NXFER - 1) % NBUF], out_b.at[pl.ds(0, CH)],
                          osem[(NXFER - 1) % NBUF]).wait()


def kernel(tokens, wte_weight, shared_prompt, u, v):
    idx = tokens.reshape(B * T).astype(jnp.int32)
    u16 = jnp.tile(u.reshape(NT, 1), (1, LANES)).reshape(NT * LANES)
    out = _mpt_sc(idx, wte_weight, shared_prompt.reshape(NT * D), u16,
                  v.reshape(D))
    return out[:, None]
